# Initial kernel scaffold; baseline (speedup 1.0000x reference)
import jax
import jax.numpy as jnp
from jax.experimental import pallas as pl


def kernel(x, edge_index, batch, W0, b0, W1, b1, W2, b2, M1, mb1, M2, mb2):
    raise NotImplementedError("write your pallas kernel here")


# speedup vs baseline: 1.0865x; 1.0865x over previous
import functools

import jax
import jax.numpy as jnp
from jax.experimental import pallas as pl
from jax.experimental.pallas import tpu as pltpu

N, E, D, H, G = 10000, 320000, 128, 128, 16
RB = 400
NBLK = N // RB


def _mm2_body(x_ref, w0_ref, b0_ref, w1_ref, b1_ref, h_ref, m_ref):
    h = jnp.dot(x_ref[...], w0_ref[...], preferred_element_type=jnp.float32) + b0_ref[...]
    h_ref[...] = h
    m_ref[...] = jnp.dot(h, w1_ref[...], preferred_element_type=jnp.float32) + b1_ref[...]


def _mm2(x, W0, b0, W1, b1):
    return pl.pallas_call(
        _mm2_body,
        grid=(NBLK,),
        in_specs=[
            pl.BlockSpec((RB, D), lambda i: (i, 0)),
            pl.BlockSpec((D, H), lambda i: (0, 0)),
            pl.BlockSpec((1, H), lambda i: (0, 0)),
            pl.BlockSpec((H, H), lambda i: (0, 0)),
            pl.BlockSpec((1, H), lambda i: (0, 0)),
        ],
        out_specs=[
            pl.BlockSpec((RB, H), lambda i: (i, 0)),
            pl.BlockSpec((RB, H), lambda i: (i, 0)),
        ],
        out_shape=[
            jax.ShapeDtypeStruct((N, H), jnp.float32),
            jax.ShapeDtypeStruct((N, H), jnp.float32),
        ],
    )(x, W0, b0.reshape(1, H), W1, b1.reshape(1, H))


def kernel(x, edge_index, batch, W0, b0, W1, b1, W2, b2, M1, mb1, M2, mb2):
    n = x.shape[0]
    loop = jnp.arange(n, dtype=edge_index.dtype)
    src = jnp.concatenate([edge_index[0], loop])
    dst = jnp.concatenate([edge_index[1], loop])
    deg = jnp.zeros((n,), x.dtype).at[dst].add(1.0)
    dinv = 1.0 / jnp.sqrt(jnp.maximum(deg, 1.0))
    norm = dinv[src] * dinv[dst]
    h, m = _mm2(x, W0, b0, W1, b1)

    def gcn_agg(m, h):
        msg = m[src] * norm[:, None]
        agg = jnp.zeros_like(m).at[dst].add(msg)
        return jax.nn.relu(agg) + h

    h = gcn_agg(m, h)
    m2 = h @ W2 + b2
    h = gcn_agg(m2, h)
    sums = jax.ops.segment_sum(h, batch, num_segments=G)
    cnt = jax.ops.segment_sum(jnp.ones((n, 1), h.dtype), batch, num_segments=G)
    pooled = sums / jnp.maximum(cnt, 1.0)
    hid = jax.nn.relu(pooled @ M1 + mb1)
    out = hid @ M2 + mb2
    return out


# speedup vs baseline: 6.4635x; 5.9489x over previous
import functools

import jax
import jax.numpy as jnp
from jax import lax
from jax.experimental import pallas as pl
from jax.experimental.pallas import tpu as pltpu
from jax.experimental.pallas import tpu_sc as plsc

N, E, D, H, G = 10000, 320000, 128, 128, 16
NP = 10240
NC, NS = 2, 16
TILES = NC * NS
CH = 128
EPT = 10240
CPT = EPT // CH
ZR = NP // NS
RB = 512
NBLK = NP // RB

_mesh = plsc.VectorSubcoreMesh(core_axis_name="c", subcore_axis_name="s")


def _sc_deg(dst3, ones_rows, zero_rows):
    @functools.partial(
        pl.kernel,
        out_type=jax.ShapeDtypeStruct((NC, NP, D), jnp.float32),
        mesh=_mesh,
        scratch_types=[
            pltpu.VMEM((CH,), jnp.int32),
            pltpu.VMEM((CH, D), jnp.float32),
            pltpu.VMEM_SHARED((NP, D), jnp.float32),
        ],
    )
    def k(dst_hbm, ones_hbm, zz_hbm, out_hbm, didx, ones_v, deg_sh):
        cid = lax.axis_index("c")
        sid = lax.axis_index("s")
        wid = cid * NS + sid
        pltpu.sync_copy(zz_hbm, deg_sh.at[pl.ds(sid * ZR, ZR)])
        pltpu.sync_copy(ones_hbm, ones_v)
        plsc.subcore_barrier()

        def body(ch, carry):
            pltpu.sync_copy(dst_hbm.at[wid, ch], didx)
            pltpu.sync_copy(ones_v, deg_sh.at[didx], add=True)
            return carry

        lax.fori_loop(0, CPT, body, 0)
        plsc.subcore_barrier()
        pltpu.sync_copy(deg_sh.at[pl.ds(sid * ZR, ZR)],
                        out_hbm.at[cid, pl.ds(sid * ZR, ZR)])

    return k(dst3, ones_rows, zero_rows)


def _sc_scatter(mt, src3, dst3, zero_rows):
    @functools.partial(
        pl.kernel,
        out_type=jax.ShapeDtypeStruct((NC, NP, D), jnp.float32),
        mesh=_mesh,
        scratch_types=[
            pltpu.VMEM((CH,), jnp.int32),
            pltpu.VMEM((CH,), jnp.int32),
            pltpu.VMEM((CH, D), jnp.float32),
            pltpu.VMEM_SHARED((NP, D), jnp.float32),
            pltpu.SemaphoreType.DMA,
        ],
    )
    def k(mt_hbm, src_hbm, dst_hbm, zz_hbm, out_hbm,
          sidx, didx, rows, agg_sh, sem):
        cid = lax.axis_index("c")
        sid = lax.axis_index("s")
        wid = cid * NS + sid
        pltpu.sync_copy(zz_hbm, agg_sh.at[pl.ds(sid * ZR, ZR)])
        plsc.subcore_barrier()

        def body(ch, carry):
            pltpu.sync_copy(src_hbm.at[wid, ch], sidx)
            pltpu.sync_copy(dst_hbm.at[wid, ch], didx)
            pltpu.async_copy(mt_hbm.at[sidx], rows, sem).wait()
            pltpu.sync_copy(rows, agg_sh.at[didx], add=True)
            return carry

        lax.fori_loop(0, CPT, body, 0)
        plsc.subcore_barrier()
        pltpu.sync_copy(agg_sh.at[pl.ds(sid * ZR, ZR)],
                        out_hbm.at[cid, pl.ds(sid * ZR, ZR)])

    return k(mt, src3, dst3, zero_rows)


def _mm2_body(x_ref, w0_ref, b0_ref, w1_ref, b1_ref, h_ref, m_ref):
    h = jnp.dot(x_ref[...], w0_ref[...],
                preferred_element_type=jnp.float32) + b0_ref[...]
    h_ref[...] = h
    m_ref[...] = jnp.dot(h, w1_ref[...],
                         preferred_element_type=jnp.float32) + b1_ref[...]


def _mm2(x, W0, b0, W1, b1):
    return pl.pallas_call(
        _mm2_body,
        grid=(NBLK,),
        in_specs=[
            pl.BlockSpec((RB, D), lambda i: (i, 0)),
            pl.BlockSpec((D, H), lambda i: (0, 0)),
            pl.BlockSpec((1, H), lambda i: (0, 0)),
            pl.BlockSpec((H, H), lambda i: (0, 0)),
            pl.BlockSpec((1, H), lambda i: (0, 0)),
        ],
        out_specs=[
            pl.BlockSpec((RB, H), lambda i: (i, 0)),
            pl.BlockSpec((RB, H), lambda i: (i, 0)),
        ],
        out_shape=[
            jax.ShapeDtypeStruct((NP, H), jnp.float32),
            jax.ShapeDtypeStruct((NP, H), jnp.float32),
        ],
    )(x, W0, b0.reshape(1, H), W1, b1.reshape(1, H))


def _scale_body(m_ref, d0_ref, d1_ref, o_ref):
    dinv = lax.rsqrt(d0_ref[...] + d1_ref[...] + 1.0)
    o_ref[...] = m_ref[...] * dinv


def _scale(m, d0, d1):
    return pl.pallas_call(
        _scale_body,
        grid=(NBLK,),
        in_specs=[
            pl.BlockSpec((RB, H), lambda i: (i, 0)),
            pl.BlockSpec((RB, 1), lambda i: (i, 0)),
            pl.BlockSpec((RB, 1), lambda i: (i, 0)),
        ],
        out_specs=pl.BlockSpec((RB, H), lambda i: (i, 0)),
        out_shape=jax.ShapeDtypeStruct((NP, H), jnp.float32),
    )(m, d0, d1)


def _combine_body(s0_ref, s1_ref, mt_ref, hp_ref, d0_ref, d1_ref,
                  w_ref, b_ref, h_ref, mt2_ref):
    dinv = lax.rsqrt(d0_ref[...] + d1_ref[...] + 1.0)
    agg = (s0_ref[...] + s1_ref[...] + mt_ref[...]) * dinv
    h = jnp.maximum(agg, 0.0) + hp_ref[...]
    h_ref[...] = h
    mt2_ref[...] = (jnp.dot(h, w_ref[...],
                            preferred_element_type=jnp.float32)
                    + b_ref[...]) * dinv


def _combine(s0, s1, mt, hprev, d0, d1, W, b):
    return pl.pallas_call(
        _combine_body,
        grid=(NBLK,),
        in_specs=[
            pl.BlockSpec((RB, H), lambda i: (i, 0)),
            pl.BlockSpec((RB, H), lambda i: (i, 0)),
            pl.BlockSpec((RB, H), lambda i: (i, 0)),
            pl.BlockSpec((RB, H), lambda i: (i, 0)),
            pl.BlockSpec((RB, 1), lambda i: (i, 0)),
            pl.BlockSpec((RB, 1), lambda i: (i, 0)),
            pl.BlockSpec((H, H), lambda i: (0, 0)),
            pl.BlockSpec((1, H), lambda i: (0, 0)),
        ],
        out_specs=[
            pl.BlockSpec((RB, H), lambda i: (i, 0)),
            pl.BlockSpec((RB, H), lambda i: (i, 0)),
        ],
        out_shape=[
            jax.ShapeDtypeStruct((NP, H), jnp.float32),
            jax.ShapeDtypeStruct((NP, H), jnp.float32),
        ],
    )(s0, s1, mt, hprev, d0, d1, W, b.reshape(1, H))


def _final_body(s0_ref, s1_ref, mt_ref, hp_ref, d0_ref, d1_ref, bc_ref,
                m1_ref, mb1_ref, m2_ref, mb2_ref, out_ref, sums, cnts):
    i = pl.program_id(0)

    @pl.when(i == 0)
    def _():
        sums[...] = jnp.zeros_like(sums)
        cnts[...] = jnp.zeros_like(cnts)

    dinv = lax.rsqrt(d0_ref[...] + d1_ref[...] + 1.0)
    agg = (s0_ref[...] + s1_ref[...] + mt_ref[...]) * dinv
    h2 = jnp.maximum(agg, 0.0) + hp_ref[...]
    gids = lax.broadcasted_iota(jnp.int32, (G, RB), 0)
    oh = jnp.where(bc_ref[0] == gids, 1.0, 0.0)
    sums[...] += jnp.dot(oh, h2, preferred_element_type=jnp.float32)
    cnts[...] += jnp.dot(oh, jnp.ones((RB, H), jnp.float32),
                         preferred_element_type=jnp.float32)

    @pl.when(i == NBLK - 1)
    def _():
        pooled = sums[...] / jnp.maximum(cnts[...], 1.0)
        hid = jnp.maximum(
            jnp.dot(pooled, m1_ref[...],
                    preferred_element_type=jnp.float32) + mb1_ref[...], 0.0)
        out_ref[...] = jnp.dot(hid, m2_ref[...],
                               preferred_element_type=jnp.float32) + mb2_ref[...]


def _final(s0, s1, mt, hprev, d0, d1, batch2, M1, mb1, M2, mb2):
    return pl.pallas_call(
        _final_body,
        grid=(NBLK,),
        in_specs=[
            pl.BlockSpec((RB, H), lambda i: (i, 0)),
            pl.BlockSpec((RB, H), lambda i: (i, 0)),
            pl.BlockSpec((RB, H), lambda i: (i, 0)),
            pl.BlockSpec((RB, H), lambda i: (i, 0)),
            pl.BlockSpec((RB, 1), lambda i: (i, 0)),
            pl.BlockSpec((RB, 1), lambda i: (i, 0)),
            pl.BlockSpec((1, 1, RB), lambda i: (i, 0, 0)),
            pl.BlockSpec((H, 256), lambda i: (0, 0)),
            pl.BlockSpec((1, 256), lambda i: (0, 0)),
            pl.BlockSpec((256, 1), lambda i: (0, 0)),
            pl.BlockSpec((1, 1), lambda i: (0, 0)),
        ],
        out_specs=pl.BlockSpec((G, 1), lambda i: (0, 0)),
        out_shape=jax.ShapeDtypeStruct((G, 1), jnp.float32),
        scratch_shapes=[
            pltpu.VMEM((G, H), jnp.float32),
            pltpu.VMEM((G, H), jnp.float32),
        ],
    )(s0, s1, mt, hprev, d0, d1, batch2, M1, mb1.reshape(1, 256), M2,
      mb2.reshape(1, 1))


def kernel(x, edge_index, batch, W0, b0, W1, b1, W2, b2, M1, mb1, M2, mb2):
    ep = TILES * EPT - E
    src3 = jnp.concatenate(
        [edge_index[0], jnp.zeros((ep,), jnp.int32)]).reshape(TILES, CPT, CH)
    dst3 = jnp.concatenate(
        [edge_index[1], jnp.full((ep,), N, jnp.int32)]).reshape(TILES, CPT, CH)
    x_p = jnp.pad(x, ((0, NP - N), (0, 0)))
    batch2 = jnp.concatenate(
        [batch, jnp.full((NP - N,), G, jnp.int32)]).reshape(NBLK, 1, RB)

    ones128 = jnp.ones((CH, D), jnp.float32)
    zro = jnp.zeros((ZR, D), jnp.float32)

    degp = _sc_deg(dst3, ones128, zro)
    d0 = degp[0, :, 0:1]
    d1 = degp[1, :, 0:1]

    h0, m1 = _mm2(x_p, W0, b0, W1, b1)
    mt1 = _scale(m1, d0, d1)
    s1 = _sc_scatter(mt1, src3, dst3, zro)
    h1, mt2 = _combine(s1[0], s1[1], mt1, h0, d0, d1, W2, b2)
    s2 = _sc_scatter(mt2, src3, dst3, zro)
    return _final(s2[0], s2[1], mt2, h1, d0, d1, batch2, M1, mb1, M2, mb2)


# speedup vs baseline: 7.9582x; 1.2313x over previous
import functools

import jax
import jax.numpy as jnp
from jax import lax
from jax.experimental import pallas as pl
from jax.experimental.pallas import tpu as pltpu
from jax.experimental.pallas import tpu_sc as plsc

N, E, D, H, G = 10000, 320000, 128, 128, 16
NP = 10240
NC, NS = 2, 16
TILES = NC * NS
CH = 128
CPT = 80
EPT = CPT * CH
ZR = NP // NS
NB = 2
NQ = 4
RB = 512
NBLK = NP // RB

_mesh = plsc.VectorSubcoreMesh(core_axis_name="c", subcore_axis_name="s")


def _sc_deg(dst3, ones_rows, zero_rows):
    @functools.partial(
        pl.kernel,
        out_type=jax.ShapeDtypeStruct((NC, NP, D), jnp.float32),
        mesh=_mesh,
        scratch_types=[
            pltpu.VMEM((CPT, CH), jnp.int32),
            pltpu.VMEM((CH, D), jnp.float32),
            pltpu.VMEM_SHARED((NP, D), jnp.float32),
        ],
    )
    def k(dst_hbm, ones_hbm, zz_hbm, out_hbm, didx, ones_v, deg_sh):
        cid = lax.axis_index("c")
        sid = lax.axis_index("s")
        wid = cid * NS + sid
        pltpu.sync_copy(zz_hbm, deg_sh.at[pl.ds(sid * ZR, ZR)])
        pltpu.sync_copy(ones_hbm, ones_v)
        pltpu.sync_copy(dst_hbm.at[wid], didx)
        plsc.subcore_barrier()

        def body(ch, carry):
            pltpu.sync_copy(ones_v, deg_sh.at[didx.at[ch]], add=True)
            return carry

        lax.fori_loop(0, CPT, body, 0)
        plsc.subcore_barrier()
        pltpu.sync_copy(deg_sh.at[pl.ds(sid * ZR, ZR)],
                        out_hbm.at[cid, pl.ds(sid * ZR, ZR)])

    return k(dst3, ones_rows, zero_rows)


def _sc_scatter(mt, src3, dst3, zero_rows):
    @functools.partial(
        pl.kernel,
        out_type=jax.ShapeDtypeStruct((NC, NP, D), jnp.float32),
        mesh=_mesh,
        scratch_types=[
            pltpu.VMEM((NQ, CH), jnp.int32),
            pltpu.VMEM((NQ, CH), jnp.int32),
            pltpu.VMEM((NB, CH, D), jnp.float32),
            pltpu.VMEM_SHARED((NP, D), jnp.float32),
            pltpu.SemaphoreType.DMA,
            pltpu.SemaphoreType.DMA,
        ],
    )
    def k(mt_hbm, src_hbm, dst_hbm, zz_hbm, out_hbm,
          sidx, didx, rows, agg_sh, gsem, isem):
        cid = lax.axis_index("c")
        sid = lax.axis_index("s")
        wid = cid * NS + sid
        pltpu.sync_copy(zz_hbm, agg_sh.at[pl.ds(sid * ZR, ZR)])

        def fetch_idx(ch):
            q = lax.rem(ch, NQ)
            pltpu.async_copy(src_hbm.at[wid, ch], sidx.at[q], isem)
            pltpu.async_copy(dst_hbm.at[wid, ch], didx.at[q], isem)

        def wait_idx():
            pltpu.make_async_copy(src_hbm.at[0, 0], sidx.at[0], isem).wait()
            pltpu.make_async_copy(src_hbm.at[0, 0], didx.at[0], isem).wait()

        def start_gather(ch, b):
            q = lax.rem(ch, NQ)
            pltpu.async_copy(mt_hbm.at[sidx.at[q]], rows.at[b], gsem)

        plsc.subcore_barrier()

        for ch in range(NQ):
            fetch_idx(ch)
        for ch in range(NB):
            wait_idx()
            start_gather(ch, ch)

        def body(ch, carry):
            b = lax.rem(ch, NB)
            q = lax.rem(ch, NQ)
            pltpu.make_async_copy(mt_hbm.at[pl.ds(0, CH)],
                                  rows.at[b], gsem).wait()
            pltpu.sync_copy(rows.at[b], agg_sh.at[didx.at[q]], add=True)

            @pl.when(ch + NQ < CPT)
            def _():
                fetch_idx(ch + NQ)

            @pl.when(ch + NB < CPT)
            def _():
                wait_idx()
                start_gather(ch + NB, b)

            return carry

        lax.fori_loop(0, CPT, body, 0)
        plsc.subcore_barrier()
        pltpu.sync_copy(agg_sh.at[pl.ds(sid * ZR, ZR)],
                        out_hbm.at[cid, pl.ds(sid * ZR, ZR)])

    return k(mt, src3, dst3, zero_rows)


def _mm2_body(x_ref, w0_ref, b0_ref, w1_ref, b1_ref, h_ref, m_ref):
    h = jnp.dot(x_ref[...], w0_ref[...],
                preferred_element_type=jnp.float32) + b0_ref[...]
    h_ref[...] = h
    m_ref[...] = jnp.dot(h, w1_ref[...],
                         preferred_element_type=jnp.float32) + b1_ref[...]


def _mm2(x, W0, b0, W1, b1):
    return pl.pallas_call(
        _mm2_body,
        grid=(NBLK,),
        in_specs=[
            pl.BlockSpec((RB, D), lambda i: (i, 0)),
            pl.BlockSpec((D, H), lambda i: (0, 0)),
            pl.BlockSpec((1, H), lambda i: (0, 0)),
            pl.BlockSpec((H, H), lambda i: (0, 0)),
            pl.BlockSpec((1, H), lambda i: (0, 0)),
        ],
        out_specs=[
            pl.BlockSpec((RB, H), lambda i: (i, 0)),
            pl.BlockSpec((RB, H), lambda i: (i, 0)),
        ],
        out_shape=[
            jax.ShapeDtypeStruct((NP, H), jnp.float32),
            jax.ShapeDtypeStruct((NP, H), jnp.float32),
        ],
    )(x, W0, b0.reshape(1, H), W1, b1.reshape(1, H))


def _scale_body(m_ref, d0_ref, d1_ref, o_ref):
    dinv = lax.rsqrt(d0_ref[...] + d1_ref[...] + 1.0)
    o_ref[...] = m_ref[...] * dinv


def _scale(m, d0, d1):
    return pl.pallas_call(
        _scale_body,
        grid=(NBLK,),
        in_specs=[
            pl.BlockSpec((RB, H), lambda i: (i, 0)),
            pl.BlockSpec((RB, 1), lambda i: (i, 0)),
            pl.BlockSpec((RB, 1), lambda i: (i, 0)),
        ],
        out_specs=pl.BlockSpec((RB, H), lambda i: (i, 0)),
        out_shape=jax.ShapeDtypeStruct((NP, H), jnp.float32),
    )(m, d0, d1)


def _combine_body(s0_ref, s1_ref, mt_ref, hp_ref, d0_ref, d1_ref,
                  w_ref, b_ref, h_ref, mt2_ref):
    dinv = lax.rsqrt(d0_ref[...] + d1_ref[...] + 1.0)
    agg = (s0_ref[...] + s1_ref[...] + mt_ref[...]) * dinv
    h = jnp.maximum(agg, 0.0) + hp_ref[...]
    h_ref[...] = h
    mt2_ref[...] = (jnp.dot(h, w_ref[...],
                            preferred_element_type=jnp.float32)
                    + b_ref[...]) * dinv


def _combine(s0, s1, mt, hprev, d0, d1, W, b):
    return pl.pallas_call(
        _combine_body,
        grid=(NBLK,),
        in_specs=[
            pl.BlockSpec((RB, H), lambda i: (i, 0)),
            pl.BlockSpec((RB, H), lambda i: (i, 0)),
            pl.BlockSpec((RB, H), lambda i: (i, 0)),
            pl.BlockSpec((RB, H), lambda i: (i, 0)),
            pl.BlockSpec((RB, 1), lambda i: (i, 0)),
            pl.BlockSpec((RB, 1), lambda i: (i, 0)),
            pl.BlockSpec((H, H), lambda i: (0, 0)),
            pl.BlockSpec((1, H), lambda i: (0, 0)),
        ],
        out_specs=[
            pl.BlockSpec((RB, H), lambda i: (i, 0)),
            pl.BlockSpec((RB, H), lambda i: (i, 0)),
        ],
        out_shape=[
            jax.ShapeDtypeStruct((NP, H), jnp.float32),
            jax.ShapeDtypeStruct((NP, H), jnp.float32),
        ],
    )(s0, s1, mt, hprev, d0, d1, W, b.reshape(1, H))


def _final_body(s0_ref, s1_ref, mt_ref, hp_ref, d0_ref, d1_ref, bc_ref,
                m1_ref, mb1_ref, m2_ref, mb2_ref, out_ref, sums, cnts):
    i = pl.program_id(0)

    @pl.when(i == 0)
    def _():
        sums[...] = jnp.zeros_like(sums)
        cnts[...] = jnp.zeros_like(cnts)

    dinv = lax.rsqrt(d0_ref[...] + d1_ref[...] + 1.0)
    agg = (s0_ref[...] + s1_ref[...] + mt_ref[...]) * dinv
    h2 = jnp.maximum(agg, 0.0) + hp_ref[...]
    gids = lax.broadcasted_iota(jnp.int32, (G, RB), 0)
    oh = jnp.where(bc_ref[0] == gids, 1.0, 0.0)
    sums[...] += jnp.dot(oh, h2, preferred_element_type=jnp.float32)
    cnts[...] += jnp.dot(oh, jnp.ones((RB, H), jnp.float32),
                         preferred_element_type=jnp.float32)

    @pl.when(i == NBLK - 1)
    def _():
        pooled = sums[...] / jnp.maximum(cnts[...], 1.0)
        hid = jnp.maximum(
            jnp.dot(pooled, m1_ref[...],
                    preferred_element_type=jnp.float32) + mb1_ref[...], 0.0)
        out_ref[...] = jnp.dot(hid, m2_ref[...],
                               preferred_element_type=jnp.float32) + mb2_ref[...]


def _final(s0, s1, mt, hprev, d0, d1, batch2, M1, mb1, M2, mb2):
    return pl.pallas_call(
        _final_body,
        grid=(NBLK,),
        in_specs=[
            pl.BlockSpec((RB, H), lambda i: (i, 0)),
            pl.BlockSpec((RB, H), lambda i: (i, 0)),
            pl.BlockSpec((RB, H), lambda i: (i, 0)),
            pl.BlockSpec((RB, H), lambda i: (i, 0)),
            pl.BlockSpec((RB, 1), lambda i: (i, 0)),
            pl.BlockSpec((RB, 1), lambda i: (i, 0)),
            pl.BlockSpec((1, 1, RB), lambda i: (i, 0, 0)),
            pl.BlockSpec((H, 256), lambda i: (0, 0)),
            pl.BlockSpec((1, 256), lambda i: (0, 0)),
            pl.BlockSpec((256, 1), lambda i: (0, 0)),
            pl.BlockSpec((1, 1), lambda i: (0, 0)),
        ],
        out_specs=pl.BlockSpec((G, 1), lambda i: (0, 0)),
        out_shape=jax.ShapeDtypeStruct((G, 1), jnp.float32),
        scratch_shapes=[
            pltpu.VMEM((G, H), jnp.float32),
            pltpu.VMEM((G, H), jnp.float32),
        ],
    )(s0, s1, mt, hprev, d0, d1, batch2, M1, mb1.reshape(1, 256), M2,
      mb2.reshape(1, 1))


def kernel(x, edge_index, batch, W0, b0, W1, b1, W2, b2, M1, mb1, M2, mb2):
    ep = TILES * EPT - E
    src3 = jnp.concatenate(
        [edge_index[0], jnp.zeros((ep,), jnp.int32)]).reshape(TILES, CPT, CH)
    dst3 = jnp.concatenate(
        [edge_index[1], jnp.full((ep,), N, jnp.int32)]).reshape(TILES, CPT, CH)
    x_p = jnp.pad(x, ((0, NP - N), (0, 0)))
    batch2 = jnp.concatenate(
        [batch, jnp.full((NP - N,), G, jnp.int32)]).reshape(NBLK, 1, RB)

    ones128 = jnp.ones((CH, D), jnp.float32)
    zro = jnp.zeros((ZR, D), jnp.float32)

    degp = _sc_deg(dst3, ones128, zro)
    d0 = degp[0, :, 0:1]
    d1 = degp[1, :, 0:1]

    h0, m1 = _mm2(x_p, W0, b0, W1, b1)
    mt1 = _scale(m1, d0, d1)
    s1 = _sc_scatter(mt1, src3, dst3, zro)
    h1, mt2 = _combine(s1[0], s1[1], mt1, h0, d0, d1, W2, b2)
    s2 = _sc_scatter(mt2, src3, dst3, zro)
    return _final(s2[0], s2[1], mt2, h1, d0, d1, batch2, M1, mb1, M2, mb2)


# speedup vs baseline: 24.7791x; 3.1137x over previous
import functools

import jax
import jax.numpy as jnp
from jax import lax
from jax.experimental import pallas as pl
from jax.experimental.pallas import tpu as pltpu
from jax.experimental.pallas import tpu_sc as plsc

N, E, D, H, G = 10000, 320000, 128, 128, 16
NP = 10240
NC, NS = 2, 16
TILES = NC * NS
CH = 128
CPT = 80
EPT = CPT * CH
ZR = NP // NS
NB = 2
NQ = 4
RB = 512
NBLK = NP // RB

_mesh = plsc.VectorSubcoreMesh(core_axis_name="c", subcore_axis_name="s")


def _sc_deg(dst3, ones_rows, zero_rows):
    @functools.partial(
        pl.kernel,
        out_type=jax.ShapeDtypeStruct((NC, NP, D), jnp.float32),
        mesh=_mesh,
        scratch_types=[
            pltpu.VMEM((CPT, CH), jnp.int32),
            pltpu.VMEM((CH, D), jnp.float32),
            pltpu.VMEM_SHARED((NP, D), jnp.float32),
        ],
    )
    def k(dst_hbm, ones_hbm, zz_hbm, out_hbm, didx, ones_v, deg_sh):
        cid = lax.axis_index("c")
        sid = lax.axis_index("s")
        wid = cid * NS + sid
        nch = jnp.minimum(CPT, (E - wid * EPT) // CH)
        pltpu.sync_copy(zz_hbm, deg_sh.at[pl.ds(sid * ZR, ZR)])
        pltpu.sync_copy(ones_hbm, ones_v)
        pltpu.sync_copy(dst_hbm.at[wid], didx)
        plsc.subcore_barrier()

        def body(ch, carry):
            pltpu.sync_copy(ones_v, deg_sh.at[didx.at[ch]], add=True)
            return carry

        lax.fori_loop(0, nch, body, 0)
        plsc.subcore_barrier()
        pltpu.sync_copy(deg_sh.at[pl.ds(sid * ZR, ZR)],
                        out_hbm.at[cid, pl.ds(sid * ZR, ZR)])

    return k(dst3, ones_rows, zero_rows)


def _sc_scatter(mt, src3, dst3, zero_rows):
    @functools.partial(
        pl.kernel,
        out_type=jax.ShapeDtypeStruct((NC, NP, D), jnp.float32),
        mesh=_mesh,
        scratch_types=[
            pltpu.VMEM((NQ, CH), jnp.int32),
            pltpu.VMEM((NQ, CH), jnp.int32),
            pltpu.VMEM((NB, CH, D), jnp.float32),
            pltpu.VMEM_SHARED((NP, D), jnp.float32),
            pltpu.SemaphoreType.DMA,
            pltpu.SemaphoreType.DMA,
        ],
    )
    def k(mt_hbm, src_hbm, dst_hbm, zz_hbm, out_hbm,
          sidx, didx, rows, agg_sh, gsem, isem):
        cid = lax.axis_index("c")
        sid = lax.axis_index("s")
        wid = cid * NS + sid
        nch = jnp.minimum(CPT, (E - wid * EPT) // CH)
        pltpu.sync_copy(zz_hbm, agg_sh.at[pl.ds(sid * ZR, ZR)])

        def fetch_idx(ch):
            q = lax.rem(ch, NQ)
            pltpu.async_copy(src_hbm.at[wid, ch], sidx.at[q], isem)
            pltpu.async_copy(dst_hbm.at[wid, ch], didx.at[q], isem)

        def wait_idx():
            pltpu.make_async_copy(src_hbm.at[0, 0], sidx.at[0], isem).wait()
            pltpu.make_async_copy(src_hbm.at[0, 0], didx.at[0], isem).wait()

        def start_gather(ch, b):
            q = lax.rem(ch, NQ)
            pltpu.async_copy(mt_hbm.at[sidx.at[q]], rows.at[b], gsem)

        plsc.subcore_barrier()

        for ch in range(NQ):
            fetch_idx(ch)
        for ch in range(NB):
            wait_idx()
            start_gather(ch, ch)

        def body(ch, carry):
            b = lax.rem(ch, NB)
            q = lax.rem(ch, NQ)
            pltpu.make_async_copy(mt_hbm.at[pl.ds(0, CH)],
                                  rows.at[b], gsem).wait()
            pltpu.sync_copy(rows.at[b], agg_sh.at[didx.at[q]], add=True)

            @pl.when(ch + NQ < nch)
            def _():
                fetch_idx(ch + NQ)

            @pl.when(ch + NB < nch)
            def _():
                wait_idx()
                start_gather(ch + NB, b)

            return carry

        lax.fori_loop(0, nch, body, 0)
        plsc.subcore_barrier()
        pltpu.sync_copy(agg_sh.at[pl.ds(sid * ZR, ZR)],
                        out_hbm.at[cid, pl.ds(sid * ZR, ZR)])

    return k(mt, src3, dst3, zero_rows)


def _mm2_body(x_ref, w0_ref, b0_ref, w1_ref, b1_ref, h_ref, m_ref):
    h = jnp.dot(x_ref[...], w0_ref[...],
                preferred_element_type=jnp.float32) + b0_ref[...]
    h_ref[...] = h
    m_ref[...] = jnp.dot(h, w1_ref[...],
                         preferred_element_type=jnp.float32) + b1_ref[...]


def _mm2(x, W0, b0, W1, b1):
    return pl.pallas_call(
        _mm2_body,
        grid=(NBLK,),
        in_specs=[
            pl.BlockSpec((RB, D), lambda i: (i, 0)),
            pl.BlockSpec((D, H), lambda i: (0, 0)),
            pl.BlockSpec((1, H), lambda i: (0, 0)),
            pl.BlockSpec((H, H), lambda i: (0, 0)),
            pl.BlockSpec((1, H), lambda i: (0, 0)),
        ],
        out_specs=[
            pl.BlockSpec((RB, H), lambda i: (i, 0)),
            pl.BlockSpec((RB, H), lambda i: (i, 0)),
        ],
        out_shape=[
            jax.ShapeDtypeStruct((NP, H), jnp.float32),
            jax.ShapeDtypeStruct((NP, H), jnp.float32),
        ],
    )(x, W0, b0.reshape(1, H), W1, b1.reshape(1, H))


def _scale_body(m_ref, d0_ref, d1_ref, o_ref):
    dinv = lax.rsqrt(d0_ref[...] + d1_ref[...] + 1.0)
    o_ref[...] = m_ref[...] * dinv


def _scale(m, d0, d1):
    return pl.pallas_call(
        _scale_body,
        grid=(NBLK,),
        in_specs=[
            pl.BlockSpec((RB, H), lambda i: (i, 0)),
            pl.BlockSpec((RB, 1), lambda i: (i, 0)),
            pl.BlockSpec((RB, 1), lambda i: (i, 0)),
        ],
        out_specs=pl.BlockSpec((RB, H), lambda i: (i, 0)),
        out_shape=jax.ShapeDtypeStruct((NP, H), jnp.float32),
    )(m, d0, d1)


def _combine_body(s0_ref, s1_ref, mt_ref, hp_ref, d0_ref, d1_ref,
                  w_ref, b_ref, h_ref, mt2_ref):
    dinv = lax.rsqrt(d0_ref[...] + d1_ref[...] + 1.0)
    agg = (s0_ref[...] + s1_ref[...] + mt_ref[...]) * dinv
    h = jnp.maximum(agg, 0.0) + hp_ref[...]
    h_ref[...] = h
    mt2_ref[...] = (jnp.dot(h, w_ref[...],
                            preferred_element_type=jnp.float32)
                    + b_ref[...]) * dinv


def _combine(s0, s1, mt, hprev, d0, d1, W, b):
    return pl.pallas_call(
        _combine_body,
        grid=(NBLK,),
        in_specs=[
            pl.BlockSpec((RB, H), lambda i: (i, 0)),
            pl.BlockSpec((RB, H), lambda i: (i, 0)),
            pl.BlockSpec((RB, H), lambda i: (i, 0)),
            pl.BlockSpec((RB, H), lambda i: (i, 0)),
            pl.BlockSpec((RB, 1), lambda i: (i, 0)),
            pl.BlockSpec((RB, 1), lambda i: (i, 0)),
            pl.BlockSpec((H, H), lambda i: (0, 0)),
            pl.BlockSpec((1, H), lambda i: (0, 0)),
        ],
        out_specs=[
            pl.BlockSpec((RB, H), lambda i: (i, 0)),
            pl.BlockSpec((RB, H), lambda i: (i, 0)),
        ],
        out_shape=[
            jax.ShapeDtypeStruct((NP, H), jnp.float32),
            jax.ShapeDtypeStruct((NP, H), jnp.float32),
        ],
    )(s0, s1, mt, hprev, d0, d1, W, b.reshape(1, H))


def _final_body(s0_ref, s1_ref, mt_ref, hp_ref, d0_ref, d1_ref, bc_ref,
                m1_ref, mb1_ref, m2_ref, mb2_ref, out_ref, sums, cnts):
    i = pl.program_id(0)

    @pl.when(i == 0)
    def _():
        sums[...] = jnp.zeros_like(sums)
        cnts[...] = jnp.zeros_like(cnts)

    dinv = lax.rsqrt(d0_ref[...] + d1_ref[...] + 1.0)
    agg = (s0_ref[...] + s1_ref[...] + mt_ref[...]) * dinv
    h2 = jnp.maximum(agg, 0.0) + hp_ref[...]
    gids = lax.broadcasted_iota(jnp.int32, (G, RB), 0)
    oh = jnp.where(bc_ref[0] == gids, 1.0, 0.0)
    sums[...] += jnp.dot(oh, h2, preferred_element_type=jnp.float32)
    cnts[...] += jnp.dot(oh, jnp.ones((RB, H), jnp.float32),
                         preferred_element_type=jnp.float32)

    @pl.when(i == NBLK - 1)
    def _():
        pooled = sums[...] / jnp.maximum(cnts[...], 1.0)
        hid = jnp.maximum(
            jnp.dot(pooled, m1_ref[...],
                    preferred_element_type=jnp.float32) + mb1_ref[...], 0.0)
        out_ref[...] = jnp.dot(hid, m2_ref[...],
                               preferred_element_type=jnp.float32) + mb2_ref[...]


def _final(s0, s1, mt, hprev, d0, d1, batch2, M1, mb1, M2, mb2):
    return pl.pallas_call(
        _final_body,
        grid=(NBLK,),
        in_specs=[
            pl.BlockSpec((RB, H), lambda i: (i, 0)),
            pl.BlockSpec((RB, H), lambda i: (i, 0)),
            pl.BlockSpec((RB, H), lambda i: (i, 0)),
            pl.BlockSpec((RB, H), lambda i: (i, 0)),
            pl.BlockSpec((RB, 1), lambda i: (i, 0)),
            pl.BlockSpec((RB, 1), lambda i: (i, 0)),
            pl.BlockSpec((1, 1, RB), lambda i: (i, 0, 0)),
            pl.BlockSpec((H, 256), lambda i: (0, 0)),
            pl.BlockSpec((1, 256), lambda i: (0, 0)),
            pl.BlockSpec((256, 1), lambda i: (0, 0)),
            pl.BlockSpec((1, 1), lambda i: (0, 0)),
        ],
        out_specs=pl.BlockSpec((G, 1), lambda i: (0, 0)),
        out_shape=jax.ShapeDtypeStruct((G, 1), jnp.float32),
        scratch_shapes=[
            pltpu.VMEM((G, H), jnp.float32),
            pltpu.VMEM((G, H), jnp.float32),
        ],
    )(s0, s1, mt, hprev, d0, d1, batch2, M1, mb1.reshape(1, 256), M2,
      mb2.reshape(1, 1))


def kernel(x, edge_index, batch, W0, b0, W1, b1, W2, b2, M1, mb1, M2, mb2):
    ep = TILES * EPT - E
    src3 = jnp.concatenate(
        [edge_index[0], jnp.zeros((ep,), jnp.int32)]).reshape(TILES, CPT, CH)
    dst3 = jnp.concatenate(
        [edge_index[1], jnp.full((ep,), N, jnp.int32)]).reshape(TILES, CPT, CH)
    x_p = jnp.pad(x, ((0, NP - N), (0, 0)))
    batch2 = jnp.concatenate(
        [batch, jnp.full((NP - N,), G, jnp.int32)]).reshape(NBLK, 1, RB)

    ones128 = jnp.ones((CH, D), jnp.float32)
    zro = jnp.zeros((ZR, D), jnp.float32)

    degp = _sc_deg(dst3, ones128, zro)
    d0 = degp[0, :, 0:1]
    d1 = degp[1, :, 0:1]

    h0, m1 = _mm2(x_p, W0, b0, W1, b1)
    mt1 = _scale(m1, d0, d1)
    s1 = _sc_scatter(mt1, src3, dst3, zro)
    h1, mt2 = _combine(s1[0], s1[1], mt1, h0, d0, d1, W2, b2)
    s2 = _sc_scatter(mt2, src3, dst3, zro)
    return _final(s2[0], s2[1], mt2, h1, d0, d1, batch2, M1, mb1, M2, mb2)


# speedup vs baseline: 28.8488x; 1.1642x over previous
import functools

import jax
import jax.numpy as jnp
from jax import lax
from jax.experimental import pallas as pl
from jax.experimental.pallas import tpu as pltpu
from jax.experimental.pallas import tpu_sc as plsc

N, E, D, H, G = 10000, 320000, 128, 128, 16
NP = 10240
NC, NS = 2, 16
TILES = NC * NS
CH = 128
CPT = 80
EPT = CPT * CH
ZR = NP // NS
NB = 2
NQ = 4
RB = 512
NBLK = NP // RB

_mesh = plsc.VectorSubcoreMesh(core_axis_name="c", subcore_axis_name="s")


NR = NP // D


def _sc_deg(dst3, zero_rows):
    @functools.partial(
        pl.kernel,
        out_type=jax.ShapeDtypeStruct((NC, NR, D), jnp.float32),
        mesh=_mesh,
        compiler_params=pltpu.CompilerParams(needs_layout_passes=False),
        scratch_types=[
            pltpu.VMEM((CPT, CH), jnp.int32),
            pltpu.VMEM((NR, D), jnp.float32),
            pltpu.VMEM((NR,), jnp.int32),
            pltpu.VMEM_SHARED((NR, D), jnp.float32),
        ],
    )
    def k(dst_hbm, zz_hbm, out_hbm, didx, hist, rowidx, deg_sh):
        cid = lax.axis_index("c")
        sid = lax.axis_index("s")
        wid = cid * NS + sid
        nch = jnp.minimum(CPT, (E - wid * EPT) // CH)
        zr = 8
        pltpu.sync_copy(dst_hbm.at[wid], didx)

        @pl.when(sid < NR // zr)
        def _():
            pltpu.sync_copy(zz_hbm.at[pl.ds(0, zr)],
                            deg_sh.at[pl.ds(sid * zr, zr)])
        for j in range(NR // 16):
            rowidx[pl.ds(j * 16, 16)] = lax.iota(jnp.int32, 16) + j * 16

        def zbody(i, carry):
            for j in range(D // 16):
                hist[i, pl.ds(j * 16, 16)] = jnp.zeros((16,), jnp.float32)
            return carry

        lax.fori_loop(0, NR, zbody, 0)
        plsc.subcore_barrier()

        ones16 = jnp.ones((16,), jnp.float32)

        def body(ch, carry):
            for j in range(CH // 16):
                v = didx[ch, pl.ds(j * 16, 16)]
                row = jnp.right_shift(v, 7)
                col = jnp.bitwise_and(v, 127)
                plsc.addupdate_scatter(hist, [row, col], ones16)
            return carry

        lax.fori_loop(0, nch, body, 0)
        pltpu.sync_copy(hist, deg_sh.at[rowidx], add=True)
        plsc.subcore_barrier()

        @pl.when(sid < NR // zr)
        def _():
            pltpu.sync_copy(deg_sh.at[pl.ds(sid * zr, zr)],
                            out_hbm.at[cid, pl.ds(sid * zr, zr)])

    return k(dst3, zero_rows)


def _sc_scatter(mt, src3, dst3, zero_rows):
    @functools.partial(
        pl.kernel,
        out_type=jax.ShapeDtypeStruct((NC, NP, D), jnp.float32),
        mesh=_mesh,
        scratch_types=[
            pltpu.VMEM((NQ, CH), jnp.int32),
            pltpu.VMEM((NQ, CH), jnp.int32),
            pltpu.VMEM((NB, CH, D), jnp.float32),
            pltpu.VMEM_SHARED((NP, D), jnp.float32),
            pltpu.SemaphoreType.DMA,
            pltpu.SemaphoreType.DMA,
        ],
    )
    def k(mt_hbm, src_hbm, dst_hbm, zz_hbm, out_hbm,
          sidx, didx, rows, agg_sh, gsem, isem):
        cid = lax.axis_index("c")
        sid = lax.axis_index("s")
        wid = cid * NS + sid
        nch = jnp.minimum(CPT, (E - wid * EPT) // CH)
        pltpu.sync_copy(zz_hbm, agg_sh.at[pl.ds(sid * ZR, ZR)])

        def fetch_idx(ch):
            q = lax.rem(ch, NQ)
            pltpu.async_copy(src_hbm.at[wid, ch], sidx.at[q], isem)
            pltpu.async_copy(dst_hbm.at[wid, ch], didx.at[q], isem)

        def wait_idx():
            pltpu.make_async_copy(src_hbm.at[0, 0], sidx.at[0], isem).wait()
            pltpu.make_async_copy(src_hbm.at[0, 0], didx.at[0], isem).wait()

        def start_gather(ch, b):
            q = lax.rem(ch, NQ)
            pltpu.async_copy(mt_hbm.at[sidx.at[q]], rows.at[b], gsem)

        plsc.subcore_barrier()

        for ch in range(NQ):
            fetch_idx(ch)
        for ch in range(NB):
            wait_idx()
            start_gather(ch, ch)

        def body(ch, carry):
            b = lax.rem(ch, NB)
            q = lax.rem(ch, NQ)
            pltpu.make_async_copy(mt_hbm.at[pl.ds(0, CH)],
                                  rows.at[b], gsem).wait()
            pltpu.sync_copy(rows.at[b], agg_sh.at[didx.at[q]], add=True)

            @pl.when(ch + NQ < nch)
            def _():
                fetch_idx(ch + NQ)

            @pl.when(ch + NB < nch)
            def _():
                wait_idx()
                start_gather(ch + NB, b)

            return carry

        lax.fori_loop(0, nch, body, 0)
        plsc.subcore_barrier()
        pltpu.sync_copy(agg_sh.at[pl.ds(sid * ZR, ZR)],
                        out_hbm.at[cid, pl.ds(sid * ZR, ZR)])

    return k(mt, src3, dst3, zero_rows)


def _mm2_body(x_ref, w0_ref, b0_ref, w1_ref, b1_ref, h_ref, m_ref):
    h = jnp.dot(x_ref[...], w0_ref[...],
                preferred_element_type=jnp.float32) + b0_ref[...]
    h_ref[...] = h
    m_ref[...] = jnp.dot(h, w1_ref[...],
                         preferred_element_type=jnp.float32) + b1_ref[...]


def _mm2(x, W0, b0, W1, b1):
    return pl.pallas_call(
        _mm2_body,
        grid=(NBLK,),
        in_specs=[
            pl.BlockSpec((RB, D), lambda i: (i, 0)),
            pl.BlockSpec((D, H), lambda i: (0, 0)),
            pl.BlockSpec((1, H), lambda i: (0, 0)),
            pl.BlockSpec((H, H), lambda i: (0, 0)),
            pl.BlockSpec((1, H), lambda i: (0, 0)),
        ],
        out_specs=[
            pl.BlockSpec((RB, H), lambda i: (i, 0)),
            pl.BlockSpec((RB, H), lambda i: (i, 0)),
        ],
        out_shape=[
            jax.ShapeDtypeStruct((NP, H), jnp.float32),
            jax.ShapeDtypeStruct((NP, H), jnp.float32),
        ],
    )(x, W0, b0.reshape(1, H), W1, b1.reshape(1, H))


def _scale_body(m_ref, d0_ref, d1_ref, o_ref):
    dinv = lax.rsqrt(d0_ref[...] + d1_ref[...] + 1.0)
    o_ref[...] = m_ref[...] * dinv


def _scale(m, d0, d1):
    return pl.pallas_call(
        _scale_body,
        grid=(NBLK,),
        in_specs=[
            pl.BlockSpec((RB, H), lambda i: (i, 0)),
            pl.BlockSpec((RB, 1), lambda i: (i, 0)),
            pl.BlockSpec((RB, 1), lambda i: (i, 0)),
        ],
        out_specs=pl.BlockSpec((RB, H), lambda i: (i, 0)),
        out_shape=jax.ShapeDtypeStruct((NP, H), jnp.float32),
    )(m, d0, d1)


def _combine_body(s0_ref, s1_ref, mt_ref, hp_ref, d0_ref, d1_ref,
                  w_ref, b_ref, h_ref, mt2_ref):
    dinv = lax.rsqrt(d0_ref[...] + d1_ref[...] + 1.0)
    agg = (s0_ref[...] + s1_ref[...] + mt_ref[...]) * dinv
    h = jnp.maximum(agg, 0.0) + hp_ref[...]
    h_ref[...] = h
    mt2_ref[...] = (jnp.dot(h, w_ref[...],
                            preferred_element_type=jnp.float32)
                    + b_ref[...]) * dinv


def _combine(s0, s1, mt, hprev, d0, d1, W, b):
    return pl.pallas_call(
        _combine_body,
        grid=(NBLK,),
        in_specs=[
            pl.BlockSpec((RB, H), lambda i: (i, 0)),
            pl.BlockSpec((RB, H), lambda i: (i, 0)),
            pl.BlockSpec((RB, H), lambda i: (i, 0)),
            pl.BlockSpec((RB, H), lambda i: (i, 0)),
            pl.BlockSpec((RB, 1), lambda i: (i, 0)),
            pl.BlockSpec((RB, 1), lambda i: (i, 0)),
            pl.BlockSpec((H, H), lambda i: (0, 0)),
            pl.BlockSpec((1, H), lambda i: (0, 0)),
        ],
        out_specs=[
            pl.BlockSpec((RB, H), lambda i: (i, 0)),
            pl.BlockSpec((RB, H), lambda i: (i, 0)),
        ],
        out_shape=[
            jax.ShapeDtypeStruct((NP, H), jnp.float32),
            jax.ShapeDtypeStruct((NP, H), jnp.float32),
        ],
    )(s0, s1, mt, hprev, d0, d1, W, b.reshape(1, H))


def _final_body(s0_ref, s1_ref, mt_ref, hp_ref, d0_ref, d1_ref, bc_ref,
                m1_ref, mb1_ref, m2_ref, mb2_ref, out_ref, sums, cnts):
    i = pl.program_id(0)

    @pl.when(i == 0)
    def _():
        sums[...] = jnp.zeros_like(sums)
        cnts[...] = jnp.zeros_like(cnts)

    dinv = lax.rsqrt(d0_ref[...] + d1_ref[...] + 1.0)
    agg = (s0_ref[...] + s1_ref[...] + mt_ref[...]) * dinv
    h2 = jnp.maximum(agg, 0.0) + hp_ref[...]
    gids = lax.broadcasted_iota(jnp.int32, (G, RB), 0)
    oh = jnp.where(bc_ref[0] == gids, 1.0, 0.0)
    sums[...] += jnp.dot(oh, h2, preferred_element_type=jnp.float32)
    cnts[...] += jnp.dot(oh, jnp.ones((RB, H), jnp.float32),
                         preferred_element_type=jnp.float32)

    @pl.when(i == NBLK - 1)
    def _():
        pooled = sums[...] / jnp.maximum(cnts[...], 1.0)
        hid = jnp.maximum(
            jnp.dot(pooled, m1_ref[...],
                    preferred_element_type=jnp.float32) + mb1_ref[...], 0.0)
        out_ref[...] = jnp.dot(hid, m2_ref[...],
                               preferred_element_type=jnp.float32) + mb2_ref[...]


def _final(s0, s1, mt, hprev, d0, d1, batch2, M1, mb1, M2, mb2):
    return pl.pallas_call(
        _final_body,
        grid=(NBLK,),
        in_specs=[
            pl.BlockSpec((RB, H), lambda i: (i, 0)),
            pl.BlockSpec((RB, H), lambda i: (i, 0)),
            pl.BlockSpec((RB, H), lambda i: (i, 0)),
            pl.BlockSpec((RB, H), lambda i: (i, 0)),
            pl.BlockSpec((RB, 1), lambda i: (i, 0)),
            pl.BlockSpec((RB, 1), lambda i: (i, 0)),
            pl.BlockSpec((1, 1, RB), lambda i: (i, 0, 0)),
            pl.BlockSpec((H, 256), lambda i: (0, 0)),
            pl.BlockSpec((1, 256), lambda i: (0, 0)),
            pl.BlockSpec((256, 1), lambda i: (0, 0)),
            pl.BlockSpec((1, 1), lambda i: (0, 0)),
        ],
        out_specs=pl.BlockSpec((G, 1), lambda i: (0, 0)),
        out_shape=jax.ShapeDtypeStruct((G, 1), jnp.float32),
        scratch_shapes=[
            pltpu.VMEM((G, H), jnp.float32),
            pltpu.VMEM((G, H), jnp.float32),
        ],
    )(s0, s1, mt, hprev, d0, d1, batch2, M1, mb1.reshape(1, 256), M2,
      mb2.reshape(1, 1))


def kernel(x, edge_index, batch, W0, b0, W1, b1, W2, b2, M1, mb1, M2, mb2):
    ep = TILES * EPT - E
    src3 = jnp.concatenate(
        [edge_index[0], jnp.zeros((ep,), jnp.int32)]).reshape(TILES, CPT, CH)
    dst3 = jnp.concatenate(
        [edge_index[1], jnp.full((ep,), N, jnp.int32)]).reshape(TILES, CPT, CH)
    x_p = jnp.pad(x, ((0, NP - N), (0, 0)))
    batch2 = jnp.concatenate(
        [batch, jnp.full((NP - N,), G, jnp.int32)]).reshape(NBLK, 1, RB)

    zro = jnp.zeros((ZR, D), jnp.float32)

    degp = _sc_deg(dst3, zro)
    d0 = degp[0].reshape(NP, 1)
    d1 = degp[1].reshape(NP, 1)

    h0, m1 = _mm2(x_p, W0, b0, W1, b1)
    mt1 = _scale(m1, d0, d1)
    s1 = _sc_scatter(mt1, src3, dst3, zro)
    h1, mt2 = _combine(s1[0], s1[1], mt1, h0, d0, d1, W2, b2)
    s2 = _sc_scatter(mt2, src3, dst3, zro)
    return _final(s2[0], s2[1], mt2, h1, d0, d1, batch2, M1, mb1, M2, mb2)


# speedup vs baseline: 29.4246x; 1.0200x over previous
import functools

import jax
import jax.numpy as jnp
from jax import lax
from jax.experimental import pallas as pl
from jax.experimental.pallas import tpu as pltpu
from jax.experimental.pallas import tpu_sc as plsc

N, E, D, H, G = 10000, 320000, 128, 128, 16
NP = 10240
NC, NS = 2, 16
TILES = NC * NS
CH = 128
CPT = 80
EPT = CPT * CH
ZR = NP // NS
NB = 2
NQ = 4
RB = 512
NBLK = NP // RB

_mesh = plsc.VectorSubcoreMesh(core_axis_name="c", subcore_axis_name="s")


NR = NP // D


def _sc_deg(dst3, zero_rows):
    @functools.partial(
        pl.kernel,
        out_type=jax.ShapeDtypeStruct((NC, NR, D), jnp.float32),
        mesh=_mesh,
        compiler_params=pltpu.CompilerParams(needs_layout_passes=False),
        scratch_types=[
            pltpu.VMEM((CPT, CH), jnp.int32),
            pltpu.VMEM((NR, D), jnp.float32),
            pltpu.VMEM((NR,), jnp.int32),
            pltpu.VMEM_SHARED((NR, D), jnp.float32),
        ],
    )
    def k(dst_hbm, zz_hbm, out_hbm, didx, hist, rowidx, deg_sh):
        cid = lax.axis_index("c")
        sid = lax.axis_index("s")
        wid = cid * NS + sid
        nch = jnp.minimum(CPT, (E - wid * EPT) // CH)
        zr = 8
        pltpu.sync_copy(dst_hbm.at[wid], didx)

        @pl.when(sid < NR // zr)
        def _():
            pltpu.sync_copy(zz_hbm.at[pl.ds(0, zr)],
                            deg_sh.at[pl.ds(sid * zr, zr)])
        for j in range(NR // 16):
            rowidx[pl.ds(j * 16, 16)] = lax.iota(jnp.int32, 16) + j * 16

        def zbody(i, carry):
            for j in range(D // 16):
                hist[i, pl.ds(j * 16, 16)] = jnp.zeros((16,), jnp.float32)
            return carry

        lax.fori_loop(0, NR, zbody, 0)
        plsc.subcore_barrier()

        ones16 = jnp.ones((16,), jnp.float32)

        def body(ch, carry):
            for j in range(CH // 16):
                v = didx[ch, pl.ds(j * 16, 16)]
                row = jnp.right_shift(v, 7)
                col = jnp.bitwise_and(v, 127)
                plsc.addupdate_scatter(hist, [row, col], ones16)
            return carry

        lax.fori_loop(0, nch, body, 0)
        pltpu.sync_copy(hist, deg_sh.at[rowidx], add=True)
        plsc.subcore_barrier()

        @pl.when(sid < NR // zr)
        def _():
            pltpu.sync_copy(deg_sh.at[pl.ds(sid * zr, zr)],
                            out_hbm.at[cid, pl.ds(sid * zr, zr)])

    return k(dst3, zero_rows)


def _sc_scatter(mt, src3, dst3, zero_rows):
    @functools.partial(
        pl.kernel,
        out_type=jax.ShapeDtypeStruct((NC, NP, D), jnp.float32),
        mesh=_mesh,
        scratch_types=[
            pltpu.VMEM((NQ, CH), jnp.int32),
            pltpu.VMEM((NQ, CH), jnp.int32),
            pltpu.VMEM((NB, CH, D), jnp.float32),
            pltpu.VMEM_SHARED((NP, D), jnp.float32),
            pltpu.SemaphoreType.DMA,
            pltpu.SemaphoreType.DMA,
        ],
    )
    def k(mt_hbm, src_hbm, dst_hbm, zz_hbm, out_hbm,
          sidx, didx, rows, agg_sh, gsem, isem):
        cid = lax.axis_index("c")
        sid = lax.axis_index("s")
        wid = cid * NS + sid
        nch = jnp.minimum(CPT, (E - wid * EPT) // CH)
        pltpu.sync_copy(zz_hbm, agg_sh.at[pl.ds(sid * ZR, ZR)])

        def fetch_idx(ch):
            q = lax.rem(ch, NQ)
            pltpu.async_copy(src_hbm.at[wid, ch], sidx.at[q], isem)
            pltpu.async_copy(dst_hbm.at[wid, ch], didx.at[q], isem)

        def wait_idx():
            pltpu.make_async_copy(src_hbm.at[0, 0], sidx.at[0], isem).wait()
            pltpu.make_async_copy(src_hbm.at[0, 0], didx.at[0], isem).wait()

        def start_gather(ch, b):
            q = lax.rem(ch, NQ)
            pltpu.async_copy(mt_hbm.at[sidx.at[q]], rows.at[b], gsem)

        plsc.subcore_barrier()

        for ch in range(NQ):
            fetch_idx(ch)
        for ch in range(NB):
            wait_idx()
            start_gather(ch, ch)

        def body(ch, carry):
            b = lax.rem(ch, NB)
            q = lax.rem(ch, NQ)
            pltpu.make_async_copy(mt_hbm.at[pl.ds(0, CH)],
                                  rows.at[b], gsem).wait()
            pltpu.sync_copy(rows.at[b], agg_sh.at[didx.at[q]], add=True)

            @pl.when(ch + NQ < nch)
            def _():
                fetch_idx(ch + NQ)

            @pl.when(ch + NB < nch)
            def _():
                wait_idx()
                start_gather(ch + NB, b)

            return carry

        lax.fori_loop(0, nch, body, 0)
        plsc.subcore_barrier()
        pltpu.sync_copy(agg_sh.at[pl.ds(sid * ZR, ZR)],
                        out_hbm.at[cid, pl.ds(sid * ZR, ZR)])

    return k(mt, src3, dst3, zero_rows)


def _mm2_body(x_ref, w0_ref, b0_ref, w1_ref, b1_ref, d0_ref, d1_ref,
              h_ref, mt_ref):
    h = jnp.dot(x_ref[...], w0_ref[...],
                preferred_element_type=jnp.float32) + b0_ref[...]
    h_ref[...] = h
    dinv = lax.rsqrt(d0_ref[...] + d1_ref[...] + 1.0)
    mt_ref[...] = (jnp.dot(h, w1_ref[...],
                           preferred_element_type=jnp.float32)
                   + b1_ref[...]) * dinv


def _mm2(x, W0, b0, W1, b1, d0, d1):
    return pl.pallas_call(
        _mm2_body,
        grid=(NBLK,),
        in_specs=[
            pl.BlockSpec((RB, D), lambda i: (i, 0)),
            pl.BlockSpec((D, H), lambda i: (0, 0)),
            pl.BlockSpec((1, H), lambda i: (0, 0)),
            pl.BlockSpec((H, H), lambda i: (0, 0)),
            pl.BlockSpec((1, H), lambda i: (0, 0)),
            pl.BlockSpec((RB, 1), lambda i: (i, 0)),
            pl.BlockSpec((RB, 1), lambda i: (i, 0)),
        ],
        out_specs=[
            pl.BlockSpec((RB, H), lambda i: (i, 0)),
            pl.BlockSpec((RB, H), lambda i: (i, 0)),
        ],
        out_shape=[
            jax.ShapeDtypeStruct((NP, H), jnp.float32),
            jax.ShapeDtypeStruct((NP, H), jnp.float32),
        ],
    )(x, W0, b0.reshape(1, H), W1, b1.reshape(1, H), d0, d1)


def _combine_body(s0_ref, s1_ref, mt_ref, hp_ref, d0_ref, d1_ref,
                  w_ref, b_ref, h_ref, mt2_ref):
    dinv = lax.rsqrt(d0_ref[...] + d1_ref[...] + 1.0)
    agg = (s0_ref[...] + s1_ref[...] + mt_ref[...]) * dinv
    h = jnp.maximum(agg, 0.0) + hp_ref[...]
    h_ref[...] = h
    mt2_ref[...] = (jnp.dot(h, w_ref[...],
                            preferred_element_type=jnp.float32)
                    + b_ref[...]) * dinv


def _combine(s0, s1, mt, hprev, d0, d1, W, b):
    return pl.pallas_call(
        _combine_body,
        grid=(NBLK,),
        in_specs=[
            pl.BlockSpec((RB, H), lambda i: (i, 0)),
            pl.BlockSpec((RB, H), lambda i: (i, 0)),
            pl.BlockSpec((RB, H), lambda i: (i, 0)),
            pl.BlockSpec((RB, H), lambda i: (i, 0)),
            pl.BlockSpec((RB, 1), lambda i: (i, 0)),
            pl.BlockSpec((RB, 1), lambda i: (i, 0)),
            pl.BlockSpec((H, H), lambda i: (0, 0)),
            pl.BlockSpec((1, H), lambda i: (0, 0)),
        ],
        out_specs=[
            pl.BlockSpec((RB, H), lambda i: (i, 0)),
            pl.BlockSpec((RB, H), lambda i: (i, 0)),
        ],
        out_shape=[
            jax.ShapeDtypeStruct((NP, H), jnp.float32),
            jax.ShapeDtypeStruct((NP, H), jnp.float32),
        ],
    )(s0, s1, mt, hprev, d0, d1, W, b.reshape(1, H))


def _final_body(s0_ref, s1_ref, mt_ref, hp_ref, d0_ref, d1_ref, bc_ref,
                m1_ref, mb1_ref, m2_ref, mb2_ref, out_ref, sums, cnts):
    i = pl.program_id(0)

    @pl.when(i == 0)
    def _():
        sums[...] = jnp.zeros_like(sums)
        cnts[...] = jnp.zeros_like(cnts)

    dinv = lax.rsqrt(d0_ref[...] + d1_ref[...] + 1.0)
    agg = (s0_ref[...] + s1_ref[...] + mt_ref[...]) * dinv
    h2 = jnp.maximum(agg, 0.0) + hp_ref[...]
    gids = lax.broadcasted_iota(jnp.int32, (G, RB), 0)
    oh = jnp.where(bc_ref[0] == gids, 1.0, 0.0)
    sums[...] += jnp.dot(oh, h2, preferred_element_type=jnp.float32)
    cnts[...] += jnp.dot(oh, jnp.ones((RB, H), jnp.float32),
                         preferred_element_type=jnp.float32)

    @pl.when(i == NBLK - 1)
    def _():
        pooled = sums[...] / jnp.maximum(cnts[...], 1.0)
        hid = jnp.maximum(
            jnp.dot(pooled, m1_ref[...],
                    preferred_element_type=jnp.float32) + mb1_ref[...], 0.0)
        out_ref[...] = jnp.dot(hid, m2_ref[...],
                               preferred_element_type=jnp.float32) + mb2_ref[...]


def _final(s0, s1, mt, hprev, d0, d1, batch2, M1, mb1, M2, mb2):
    return pl.pallas_call(
        _final_body,
        grid=(NBLK,),
        in_specs=[
            pl.BlockSpec((RB, H), lambda i: (i, 0)),
            pl.BlockSpec((RB, H), lambda i: (i, 0)),
            pl.BlockSpec((RB, H), lambda i: (i, 0)),
            pl.BlockSpec((RB, H), lambda i: (i, 0)),
            pl.BlockSpec((RB, 1), lambda i: (i, 0)),
            pl.BlockSpec((RB, 1), lambda i: (i, 0)),
            pl.BlockSpec((1, 1, RB), lambda i: (i, 0, 0)),
            pl.BlockSpec((H, 256), lambda i: (0, 0)),
            pl.BlockSpec((1, 256), lambda i: (0, 0)),
            pl.BlockSpec((256, 1), lambda i: (0, 0)),
            pl.BlockSpec((1, 1), lambda i: (0, 0)),
        ],
        out_specs=pl.BlockSpec((G, 1), lambda i: (0, 0)),
        out_shape=jax.ShapeDtypeStruct((G, 1), jnp.float32),
        scratch_shapes=[
            pltpu.VMEM((G, H), jnp.float32),
            pltpu.VMEM((G, H), jnp.float32),
        ],
    )(s0, s1, mt, hprev, d0, d1, batch2, M1, mb1.reshape(1, 256), M2,
      mb2.reshape(1, 1))


def kernel(x, edge_index, batch, W0, b0, W1, b1, W2, b2, M1, mb1, M2, mb2):
    ep = TILES * EPT - E
    src3 = jnp.concatenate(
        [edge_index[0], jnp.zeros((ep,), jnp.int32)]).reshape(TILES, CPT, CH)
    dst3 = jnp.concatenate(
        [edge_index[1], jnp.full((ep,), N, jnp.int32)]).reshape(TILES, CPT, CH)
    x_p = jnp.pad(x, ((0, NP - N), (0, 0)))
    batch2 = jnp.concatenate(
        [batch, jnp.full((NP - N,), G, jnp.int32)]).reshape(NBLK, 1, RB)

    zro = jnp.zeros((ZR, D), jnp.float32)

    degp = _sc_deg(dst3, zro)
    d0 = degp[0].reshape(NP, 1)
    d1 = degp[1].reshape(NP, 1)

    h0, mt1 = _mm2(x_p, W0, b0, W1, b1, d0, d1)
    s1 = _sc_scatter(mt1, src3, dst3, zro)
    h1, mt2 = _combine(s1[0], s1[1], mt1, h0, d0, d1, W2, b2)
    s2 = _sc_scatter(mt2, src3, dst3, zro)
    return _final(s2[0], s2[1], mt2, h1, d0, d1, batch2, M1, mb1, M2, mb2)
